# baseline (speedup 1.0000x reference)
import functools

import jax
import jax.numpy as jnp
from jax import lax
from jax.experimental import pallas as pl
from jax.experimental.pallas import tpu as pltpu
from jax.experimental.pallas import tpu_sc as plsc

T = 2048
DM = 1024
HD = 2048
E = 8
BLK = 256
SP = 6144
NB = SP // BLK
NC = 2
NS = 16
NW = NC * NS
CH = 128
TD = T // NW


def _gating_body(x_ref, wg_ref, eids_ref, wts_ref):
    xb = x_ref[...]
    lt = jax.lax.dot_general(
        wg_ref[...], xb, (((1,), (1,)), ((), ())),
        preferred_element_type=jnp.float32)
    eidx = jax.lax.broadcasted_iota(jnp.int32, lt.shape, 0)
    m1 = jnp.max(lt, axis=0, keepdims=True)
    i1 = jnp.min(jnp.where(lt == m1, eidx, E), axis=0, keepdims=True)
    l2 = jnp.where(eidx == i1, -jnp.inf, lt)
    m2 = jnp.max(l2, axis=0, keepdims=True)
    i2 = jnp.min(jnp.where(l2 == m2, eidx, E), axis=0, keepdims=True)
    p = jnp.exp(m2 - m1)
    w1g = 1.0 / (1.0 + p)
    w2g = p / (1.0 + p)
    eids_ref[...] = jnp.concatenate([i1, i2], axis=0)
    wts_ref[...] = jnp.concatenate([w1g, w2g], axis=0)


def _gating(xf, Wg):
    blk = 256
    return pl.pallas_call(
        _gating_body,
        grid=(T // blk,),
        in_specs=[
            pl.BlockSpec((blk, DM), lambda tb: (tb, 0)),
            pl.BlockSpec((E, DM), lambda tb: (0, 0)),
        ],
        out_specs=(
            pl.BlockSpec((2, blk), lambda tb: (0, tb)),
            pl.BlockSpec((2, blk), lambda tb: (0, tb)),
        ),
        out_shape=(
            jax.ShapeDtypeStruct((2, T), jnp.int32),
            jax.ShapeDtypeStruct((2, T), jnp.float32),
        ),
    )(xf, Wg)


def _take16(vec, lane):
    dn = jax.lax.GatherDimensionNumbers(
        offset_dims=(), collapsed_slice_dims=(0,), start_index_map=(0,))
    return jax.lax.gather(
        vec, lane[:, None], dn, (1,),
        mode=jax.lax.GatherScatterMode.PROMISE_IN_BOUNDS)


def _allsum16(v):
    lane = lax.iota(jnp.int32, 16)
    for d in (8, 4, 2, 1):
        v = v + _take16(v, jnp.bitwise_xor(lane, d))
    return v


def _cumsum16(v):
    lane = lax.iota(jnp.int32, 16)
    for d in (1, 2, 4, 8):
        sh = _take16(v, jnp.maximum(lane - d, 0))
        v = v + jnp.where(lane >= d, sh, 0)
    return v


def _dispatch_body(eids_hbm, xf_hbm, xs_hbm, pos_hbm, bmap_hbm, bval_hbm,
                   eid_v, mych_v, posl_v, sidx_v, rows_v, bm_v, bv_v,
                   gsem0, gsem1, gsem2, ssem0, ssem1, ssem2):
    c = lax.axis_index("c")
    s = lax.axis_index("s")
    wid = s * NC + c
    k = wid // NS
    t0 = (wid % NS) * CH
    myvec0 = (wid % NS) * (CH // 16)
    nch = CH // 32
    gsem = (gsem0, gsem1, gsem2)
    ssem = (ssem0, ssem1, ssem2)
    g = [None] * nch
    sc = [None] * nch

    for cj in range(min(3, nch)):
        g[cj] = pltpu.async_copy(
            xf_hbm.at[pl.ds(t0 + cj * 32, 32)], rows_v.at[cj % 3],
            gsem[cj % 3])

    pltpu.sync_copy(eids_hbm, eid_v)
    pltpu.sync_copy(eids_hbm.at[k, pl.ds(t0, CH)], mych_v)

    zeros8 = tuple(jnp.zeros((16,), jnp.int32) for _ in range(E))

    def scan_row(row, lo, hi, acc):
        def body(i, a):
            v = eid_v[row, pl.ds(i * 16, 16)]
            return tuple(a[e] + jnp.where(v == e, 1, 0) for e in range(E))
        return lax.fori_loop(lo, hi, body, acc)

    nvr = T // 16
    b0 = jnp.where(k == 0, myvec0, nvr)
    b1 = jnp.where(k == 0, 0, myvec0)
    pre = scan_row(0, 0, b0, zeros8)
    pre = scan_row(1, 0, b1, pre)
    tot = scan_row(0, b0, nvr, pre)
    tot = scan_row(1, b1, nvr, tot)

    pre_c = [_allsum16(pre[e]) for e in range(E)]
    tot_c = [_allsum16(tot[e]) for e in range(E)]

    poff = []
    nblk = []
    run = jnp.zeros((16,), jnp.int32)
    for e in range(E):
        poff.append(run)
        nb_e = jnp.right_shift(tot_c[e] + (BLK - 1), 8)
        nblk.append(nb_e)
        run = run + jnp.left_shift(nb_e, 8)

    rcnt = [poff[e] + pre_c[e] for e in range(E)]
    for j in range(CH // 16):
        v = mych_v[pl.ds(j * 16, 16)]
        pos_vec = jnp.zeros((16,), jnp.int32)
        for e in range(E):
            m = v == e
            m32 = jnp.where(m, 1, 0)
            cs = _cumsum16(m32)
            pos_vec = jnp.where(m, rcnt[e] + cs - 1, pos_vec)
            rcnt[e] = rcnt[e] + _allsum16(m32)
        posl_v[pl.ds(j * 16, 16)] = pos_vec
        sidx_v[j // 2, pl.ds((j % 2) * 16, 16)] = pos_vec

    pltpu.sync_copy(posl_v, pos_hbm.at[k, pl.ds(t0, CH)])

    @pl.when(wid == 0)
    def _():
        for half in range(2):
            jv = lax.iota(jnp.int32, 16) + half * 16
            bstart = jv * BLK
            bmv = jnp.full((16,), E - 1, jnp.int32)
            bvv = jnp.zeros((16,), jnp.int32)
            for e in range(E):
                inb = (bstart >= poff[e]) & (bstart < poff[e] + nblk[e] * BLK)
                bmv = jnp.where(inb, e, bmv)
                bvv = jnp.where(inb, 1, bvv)
            bm_v[pl.ds(half * 16, 16)] = bmv
            bv_v[pl.ds(half * 16, 16)] = bvv
        pltpu.sync_copy(bm_v, bmap_hbm)
        pltpu.sync_copy(bv_v, bval_hbm)

    waited = [False] * nch
    for cj in range(nch):
        b = cj % 3
        g[cj].wait()
        sc[cj] = pltpu.async_copy(rows_v.at[b], xs_hbm.at[sidx_v.at[cj]],
                                  ssem[b])
        if cj + 3 < nch:
            sc[cj].wait()
            waited[cj] = True
            g[cj + 3] = pltpu.async_copy(
                xf_hbm.at[pl.ds(t0 + (cj + 3) * 32, 32)], rows_v.at[b],
                gsem[b])
    for cj in range(nch):
        if not waited[cj]:
            sc[cj].wait()


def _dispatch(eids, xf):
    mesh = plsc.VectorSubcoreMesh(core_axis_name="c", subcore_axis_name="s")
    fn = pl.kernel(
        _dispatch_body,
        out_type=(
            jax.ShapeDtypeStruct((SP, DM), jnp.float32),
            jax.ShapeDtypeStruct((2, T), jnp.int32),
            jax.ShapeDtypeStruct((32,), jnp.int32),
            jax.ShapeDtypeStruct((32,), jnp.int32),
        ),
        mesh=mesh,
        scratch_types=[
            pltpu.VMEM((2, T), jnp.int32),
            pltpu.VMEM((CH,), jnp.int32),
            pltpu.VMEM((CH,), jnp.int32),
            pltpu.VMEM((CH // 32, 32), jnp.int32),
            pltpu.VMEM((3, 32, DM), jnp.float32),
            pltpu.VMEM((32,), jnp.int32),
            pltpu.VMEM((32,), jnp.int32),
            pltpu.SemaphoreType.DMA,
            pltpu.SemaphoreType.DMA,
            pltpu.SemaphoreType.DMA,
            pltpu.SemaphoreType.DMA,
            pltpu.SemaphoreType.DMA,
            pltpu.SemaphoreType.DMA,
        ],
    )
    return fn(eids, xf)


def _ffn_body(bmap_ref, bval_ref, xs_ref, w1_ref, b1_ref, w2_ref, b2_ref,
              g_ref, be_ref, ys_ref):
    b = pl.program_id(0)
    h = pl.program_id(1)

    @pl.when(bval_ref[b] != 0)
    def _():
        xb = xs_ref[...]
        hh = jnp.maximum(
            jnp.dot(xb, w1_ref[0], preferred_element_type=jnp.float32)
            + b1_ref[0], 0.0)
        part = jnp.dot(hh, w2_ref[0], preferred_element_type=jnp.float32)

        @pl.when(h == 0)
        def _():
            ys_ref[...] = part

        @pl.when(h == 1)
        def _():
            y = ys_ref[...] + part + b2_ref[0]
            mu = jnp.mean(y, axis=1, keepdims=True)
            yc = y - mu
            var = jnp.mean(yc * yc, axis=1, keepdims=True)
            ys_ref[...] = yc * jax.lax.rsqrt(var + 1e-5) * g_ref[0] \
                + be_ref[0]


def _ffn(xs, bmap, bval, W1, b1, W2, b2, gamma, beta):
    grid_spec = pltpu.PrefetchScalarGridSpec(
        num_scalar_prefetch=2,
        grid=(NB, 2),
        in_specs=[
            pl.BlockSpec((BLK, DM), lambda b, h, bm, bv: (b * bv[b], 0)),
            pl.BlockSpec((1, DM, HD // 2),
                         lambda b, h, bm, bv: (bm[b], 0, h)),
            pl.BlockSpec((1, 1, HD // 2),
                         lambda b, h, bm, bv: (bm[b], 0, h)),
            pl.BlockSpec((1, HD // 2, DM),
                         lambda b, h, bm, bv: (bm[b], h, 0)),
            pl.BlockSpec((1, 1, DM), lambda b, h, bm, bv: (bm[b], 0, 0)),
            pl.BlockSpec((1, 1, DM), lambda b, h, bm, bv: (bm[b], 0, 0)),
            pl.BlockSpec((1, 1, DM), lambda b, h, bm, bv: (bm[b], 0, 0)),
        ],
        out_specs=pl.BlockSpec(
            (BLK, DM),
            lambda b, h, bm, bv: (jnp.where(bv[b] != 0, b, NB - 1), 0)),
    )
    return pl.pallas_call(
        _ffn_body,
        grid_spec=grid_spec,
        out_shape=jax.ShapeDtypeStruct((SP, DM), jnp.float32),
        compiler_params=pltpu.CompilerParams(
            vmem_limit_bytes=100 * 1024 * 1024),
    )(bmap, bval, xs, W1, b1.reshape(E, 1, HD), W2, b2.reshape(E, 1, DM),
      gamma.reshape(E, 1, DM), beta.reshape(E, 1, DM))


def _take16(vec, lane):
    dn = jax.lax.GatherDimensionNumbers(
        offset_dims=(), collapsed_slice_dims=(0,), start_index_map=(0,))
    return jax.lax.gather(
        vec, lane[:, None], dn, (1,),
        mode=jax.lax.GatherScatterMode.PROMISE_IN_BOUNDS)

def _combine_body(ys_hbm, pos_hbm, wts_hbm, out_hbm,
                  p0_v, p1_v, w0_v, w1_v, r0_v, r1_v, ob_v,
                  g0s0, g0s1, g1s0, g1s1, sts0, sts1):
    c = lax.axis_index("c")
    s = lax.axis_index("s")
    wid = s * NC + c
    t0 = wid * TD
    ck = 16
    nch = TD // ck

    pltpu.sync_copy(pos_hbm.at[0, pl.ds(t0, TD)], p0_v)
    pltpu.sync_copy(pos_hbm.at[1, pl.ds(t0, TD)], p1_v)

    g0sem = (g0s0, g0s1)
    g1sem = (g1s0, g1s1)
    stsem = (sts0, sts1)
    g0 = [None, None]
    g1 = [None, None]
    st = [None, None]

    def gstart(cj, p):
        g0[p] = pltpu.async_copy(
            ys_hbm.at[p0_v.at[pl.ds(cj * ck, ck)]], r0_v.at[p], g0sem[p])
        g1[p] = pltpu.async_copy(
            ys_hbm.at[p1_v.at[pl.ds(cj * ck, ck)]], r1_v.at[p], g1sem[p])

    gstart(0, 0)
    pltpu.sync_copy(wts_hbm.at[0, pl.ds(t0, TD)], w0_v)
    pltpu.sync_copy(wts_hbm.at[1, pl.ds(t0, TD)], w1_v)
    for cj in range(nch):
        p = cj % 2
        g0[p].wait()
        g1[p].wait()
        if cj + 1 < nch:
            gstart(cj + 1, (cj + 1) % 2)
        if st[p] is not None:
            st[p].wait()

        wv0 = w0_v[pl.ds(cj * ck, 16)]
        wv1 = w1_v[pl.ds(cj * ck, 16)]
        obp = ob_v.at[p]
        r0p = r0_v.at[p]
        r1p = r1_v.at[p]

        def tok(j, carry):
            lane = jnp.full((16,), j, jnp.int32)
            s0 = _take16(wv0, lane)
            s1 = _take16(wv1, lane)
            for v in range(DM // 16):
                sl = pl.ds(v * 16, 16)
                obp[j, sl] = s0 * r0p[j, sl] + s1 * r1p[j, sl]
            return carry

        lax.fori_loop(0, ck, tok, jnp.int32(0))
        st[p] = pltpu.async_copy(obp, out_hbm.at[pl.ds(t0 + cj * ck, ck)],
                                 stsem[p])
    st[0].wait()
    st[1].wait()


def _combine(ys, pos, wts):
    mesh = plsc.VectorSubcoreMesh(core_axis_name="c", subcore_axis_name="s")
    fn = pl.kernel(
        _combine_body,
        out_type=jax.ShapeDtypeStruct((T, DM), jnp.float32),
        mesh=mesh,
        scratch_types=[
            pltpu.VMEM((TD,), jnp.int32),
            pltpu.VMEM((TD,), jnp.int32),
            pltpu.VMEM((TD,), jnp.float32),
            pltpu.VMEM((TD,), jnp.float32),
            pltpu.VMEM((2, 16, DM), jnp.float32),
            pltpu.VMEM((2, 16, DM), jnp.float32),
            pltpu.VMEM((2, 16, DM), jnp.float32),
            pltpu.SemaphoreType.DMA,
            pltpu.SemaphoreType.DMA,
            pltpu.SemaphoreType.DMA,
            pltpu.SemaphoreType.DMA,
            pltpu.SemaphoreType.DMA,
            pltpu.SemaphoreType.DMA,
        ],
    )
    return fn(ys, pos, wts)


def kernel(x, Wg, W1, b1, W2, b2, gamma, beta):
    Bsz, Slen, D = x.shape
    xf = x.reshape(T, DM)
    eids, wts = _gating(xf, Wg)
    xs, pos, bmap, bval = _dispatch(eids, xf)
    ys = _ffn(xs, bmap, bval, W1, b1, W2, b2, gamma, beta)
    out = _combine(ys, pos, wts)
    return out.reshape(Bsz, Slen, D)


# speedup vs baseline: 1.3186x; 1.3186x over previous
import functools

import jax
import jax.numpy as jnp
from jax import lax
from jax.experimental import pallas as pl
from jax.experimental.pallas import tpu as pltpu
from jax.experimental.pallas import tpu_sc as plsc

T = 2048
DM = 1024
HD = 2048
E = 8
BLK = 256
SP = 6144
NB = SP // BLK
NC = 2
NS = 16
NW = NC * NS
CH = 128
TD = T // NW


def _gating_body(x_ref, wg_ref, eids_ref, wts_ref):
    xb = x_ref[...]
    lt = jax.lax.dot_general(
        wg_ref[...], xb, (((1,), (1,)), ((), ())),
        preferred_element_type=jnp.float32)
    eidx = jax.lax.broadcasted_iota(jnp.int32, lt.shape, 0)
    m1 = jnp.max(lt, axis=0, keepdims=True)
    i1 = jnp.min(jnp.where(lt == m1, eidx, E), axis=0, keepdims=True)
    l2 = jnp.where(eidx == i1, -jnp.inf, lt)
    m2 = jnp.max(l2, axis=0, keepdims=True)
    i2 = jnp.min(jnp.where(l2 == m2, eidx, E), axis=0, keepdims=True)
    p = jnp.exp(m2 - m1)
    w1g = 1.0 / (1.0 + p)
    w2g = p / (1.0 + p)
    eids_ref[...] = jnp.concatenate([i1, i2], axis=0)
    wts_ref[...] = jnp.concatenate([w1g, w2g], axis=0)


def _gating(xf, Wg):
    blk = 256
    return pl.pallas_call(
        _gating_body,
        grid=(T // blk,),
        in_specs=[
            pl.BlockSpec((blk, DM), lambda tb: (tb, 0)),
            pl.BlockSpec((E, DM), lambda tb: (0, 0)),
        ],
        out_specs=(
            pl.BlockSpec((2, blk), lambda tb: (0, tb)),
            pl.BlockSpec((2, blk), lambda tb: (0, tb)),
        ),
        out_shape=(
            jax.ShapeDtypeStruct((2, T), jnp.int32),
            jax.ShapeDtypeStruct((2, T), jnp.float32),
        ),
    )(xf, Wg)


def _take16(vec, lane):
    dn = jax.lax.GatherDimensionNumbers(
        offset_dims=(), collapsed_slice_dims=(0,), start_index_map=(0,))
    return jax.lax.gather(
        vec, lane[:, None], dn, (1,),
        mode=jax.lax.GatherScatterMode.PROMISE_IN_BOUNDS)


def _allsum16(v):
    lane = lax.iota(jnp.int32, 16)
    for d in (8, 4, 2, 1):
        v = v + _take16(v, jnp.bitwise_xor(lane, d))
    return v


def _cumsum16(v):
    lane = lax.iota(jnp.int32, 16)
    for d in (1, 2, 4, 8):
        sh = _take16(v, jnp.maximum(lane - d, 0))
        v = v + jnp.where(lane >= d, sh, 0)
    return v


def _dispatch_body(eids_hbm, xf_hbm, xs_hbm, pos_hbm, bmap_hbm, bval_hbm,
                   eid_v, mych_v, posl_v, sidx_v, rows_v, bm_v, bv_v,
                   gsem0, gsem1, gsem2, ssem0, ssem1, ssem2):
    c = lax.axis_index("c")
    s = lax.axis_index("s")
    wid = s * NC + c
    k = wid // NS
    t0 = (wid % NS) * CH
    myvec0 = (wid % NS) * (CH // 16)
    nch = CH // 32
    gsem = (gsem0, gsem1, gsem2)
    ssem = (ssem0, ssem1, ssem2)
    g = [None] * nch
    sc = [None] * nch

    for cj in range(min(3, nch)):
        g[cj] = pltpu.async_copy(
            xf_hbm.at[pl.ds(t0 + cj * 32, 32)], rows_v.at[cj % 3],
            gsem[cj % 3])

    pltpu.sync_copy(eids_hbm, eid_v)
    pltpu.sync_copy(eids_hbm.at[k, pl.ds(t0, CH)], mych_v)

    zeros8 = tuple(jnp.zeros((16,), jnp.int32) for _ in range(E))

    def scan_row(row, lo, hi, acc):
        def body(i, a):
            v = eid_v[row, pl.ds(i * 16, 16)]
            return tuple(a[e] + jnp.where(v == e, 1, 0) for e in range(E))
        return lax.fori_loop(lo, hi, body, acc)

    nvr = T // 16
    b0 = jnp.where(k == 0, myvec0, nvr)
    b1 = jnp.where(k == 0, 0, myvec0)
    pre = scan_row(0, 0, b0, zeros8)
    pre = scan_row(1, 0, b1, pre)
    tot = scan_row(0, b0, nvr, pre)
    tot = scan_row(1, b1, nvr, tot)

    pre_c = [_allsum16(pre[e]) for e in range(E)]
    tot_c = [_allsum16(tot[e]) for e in range(E)]

    poff = []
    nblk = []
    run = jnp.zeros((16,), jnp.int32)
    for e in range(E):
        poff.append(run)
        nb_e = jnp.right_shift(tot_c[e] + (BLK - 1), 8)
        nblk.append(nb_e)
        run = run + jnp.left_shift(nb_e, 8)

    rcnt = [poff[e] + pre_c[e] for e in range(E)]
    for j in range(CH // 16):
        v = mych_v[pl.ds(j * 16, 16)]
        pos_vec = jnp.zeros((16,), jnp.int32)
        for e in range(E):
            m = v == e
            m32 = jnp.where(m, 1, 0)
            cs = _cumsum16(m32)
            pos_vec = jnp.where(m, rcnt[e] + cs - 1, pos_vec)
            rcnt[e] = rcnt[e] + _allsum16(m32)
        posl_v[pl.ds(j * 16, 16)] = pos_vec
        sidx_v[j // 2, pl.ds((j % 2) * 16, 16)] = pos_vec

    pltpu.sync_copy(posl_v, pos_hbm.at[k, pl.ds(t0, CH)])

    @pl.when(wid == 0)
    def _():
        for half in range(2):
            jv = lax.iota(jnp.int32, 16) + half * 16
            bstart = jv * BLK
            bmv = jnp.full((16,), E - 1, jnp.int32)
            bvv = jnp.zeros((16,), jnp.int32)
            for e in range(E):
                inb = (bstart >= poff[e]) & (bstart < poff[e] + nblk[e] * BLK)
                bmv = jnp.where(inb, e, bmv)
                bvv = jnp.where(inb, 1, bvv)
            bm_v[pl.ds(half * 16, 16)] = bmv
            bv_v[pl.ds(half * 16, 16)] = bvv
        pltpu.sync_copy(bm_v, bmap_hbm)
        pltpu.sync_copy(bv_v, bval_hbm)

    waited = [False] * nch
    for cj in range(nch):
        b = cj % 3
        g[cj].wait()
        sc[cj] = pltpu.async_copy(rows_v.at[b], xs_hbm.at[sidx_v.at[cj]],
                                  ssem[b])
        if cj + 3 < nch:
            sc[cj].wait()
            waited[cj] = True
            g[cj + 3] = pltpu.async_copy(
                xf_hbm.at[pl.ds(t0 + (cj + 3) * 32, 32)], rows_v.at[b],
                gsem[b])
    for cj in range(nch):
        if not waited[cj]:
            sc[cj].wait()


def _dispatch(eids, xf):
    mesh = plsc.VectorSubcoreMesh(core_axis_name="c", subcore_axis_name="s")
    fn = pl.kernel(
        _dispatch_body,
        out_type=(
            jax.ShapeDtypeStruct((SP, DM), jnp.float32),
            jax.ShapeDtypeStruct((2, T), jnp.int32),
            jax.ShapeDtypeStruct((32,), jnp.int32),
            jax.ShapeDtypeStruct((32,), jnp.int32),
        ),
        mesh=mesh,
        scratch_types=[
            pltpu.VMEM((2, T), jnp.int32),
            pltpu.VMEM((CH,), jnp.int32),
            pltpu.VMEM((CH,), jnp.int32),
            pltpu.VMEM((CH // 32, 32), jnp.int32),
            pltpu.VMEM((3, 32, DM), jnp.float32),
            pltpu.VMEM((32,), jnp.int32),
            pltpu.VMEM((32,), jnp.int32),
            pltpu.SemaphoreType.DMA,
            pltpu.SemaphoreType.DMA,
            pltpu.SemaphoreType.DMA,
            pltpu.SemaphoreType.DMA,
            pltpu.SemaphoreType.DMA,
            pltpu.SemaphoreType.DMA,
        ],
    )
    return fn(eids, xf)


def _ffn_body(bmap_ref, bval_ref, xs_ref, w1_ref, b1_ref, w2_ref, b2_ref,
              g_ref, be_ref, ys_ref):
    b = pl.program_id(0)

    @pl.when(bval_ref[b] != 0)
    def _():
        xb = xs_ref[...]
        h = jnp.maximum(
            jnp.dot(xb, w1_ref[0], preferred_element_type=jnp.float32)
            + b1_ref[0], 0.0)
        y = jnp.dot(h, w2_ref[0], preferred_element_type=jnp.float32) \
            + b2_ref[0]
        mu = jnp.mean(y, axis=1, keepdims=True)
        yc = y - mu
        var = jnp.mean(yc * yc, axis=1, keepdims=True)
        ys_ref[...] = yc * jax.lax.rsqrt(var + 1e-5) * g_ref[0] + be_ref[0]


def _ffn(xs, bmap, bval, W1, b1, W2, b2, gamma, beta):
    grid_spec = pltpu.PrefetchScalarGridSpec(
        num_scalar_prefetch=2,
        grid=(NB,),
        in_specs=[
            pl.BlockSpec((BLK, DM), lambda b, bm, bv: (b * bv[b], 0)),
            pl.BlockSpec((1, DM, HD), lambda b, bm, bv: (bm[b], 0, 0)),
            pl.BlockSpec((1, 1, HD), lambda b, bm, bv: (bm[b], 0, 0)),
            pl.BlockSpec((1, HD, DM), lambda b, bm, bv: (bm[b], 0, 0)),
            pl.BlockSpec((1, 1, DM), lambda b, bm, bv: (bm[b], 0, 0)),
            pl.BlockSpec((1, 1, DM), lambda b, bm, bv: (bm[b], 0, 0)),
            pl.BlockSpec((1, 1, DM), lambda b, bm, bv: (bm[b], 0, 0)),
        ],
        out_specs=pl.BlockSpec(
            (BLK, DM),
            lambda b, bm, bv: (jnp.where(bv[b] != 0, b, NB - 1), 0)),
    )
    return pl.pallas_call(
        _ffn_body,
        grid_spec=grid_spec,
        out_shape=jax.ShapeDtypeStruct((SP, DM), jnp.float32),
        compiler_params=pltpu.CompilerParams(
            vmem_limit_bytes=100 * 1024 * 1024),
    )(bmap, bval, xs, W1, b1.reshape(E, 1, HD), W2, b2.reshape(E, 1, DM),
      gamma.reshape(E, 1, DM), beta.reshape(E, 1, DM))


def _take16(vec, lane):
    dn = jax.lax.GatherDimensionNumbers(
        offset_dims=(), collapsed_slice_dims=(0,), start_index_map=(0,))
    return jax.lax.gather(
        vec, lane[:, None], dn, (1,),
        mode=jax.lax.GatherScatterMode.PROMISE_IN_BOUNDS)

def _combine_body(ys_hbm, pos_hbm, wts_hbm, out_hbm,
                  p0_v, p1_v, w0_v, w1_v, r0_v, r1_v, ob_v,
                  g0s0, g0s1, g1s0, g1s1, sts0, sts1):
    c = lax.axis_index("c")
    s = lax.axis_index("s")
    wid = s * NC + c
    t0 = wid * TD
    ck = 16
    nch = TD // ck

    pltpu.sync_copy(pos_hbm.at[0, pl.ds(t0, TD)], p0_v)
    pltpu.sync_copy(pos_hbm.at[1, pl.ds(t0, TD)], p1_v)

    g0sem = (g0s0, g0s1)
    g1sem = (g1s0, g1s1)
    stsem = (sts0, sts1)
    g0 = [None, None]
    g1 = [None, None]
    st = [None, None]

    def gstart(cj, p):
        g0[p] = pltpu.async_copy(
            ys_hbm.at[p0_v.at[pl.ds(cj * ck, ck)]], r0_v.at[p], g0sem[p])
        g1[p] = pltpu.async_copy(
            ys_hbm.at[p1_v.at[pl.ds(cj * ck, ck)]], r1_v.at[p], g1sem[p])

    gstart(0, 0)
    pltpu.sync_copy(wts_hbm.at[0, pl.ds(t0, TD)], w0_v)
    pltpu.sync_copy(wts_hbm.at[1, pl.ds(t0, TD)], w1_v)
    for cj in range(nch):
        p = cj % 2
        g0[p].wait()
        g1[p].wait()
        if cj + 1 < nch:
            gstart(cj + 1, (cj + 1) % 2)
        if st[p] is not None:
            st[p].wait()

        wv0 = w0_v[pl.ds(cj * ck, 16)]
        wv1 = w1_v[pl.ds(cj * ck, 16)]
        obp = ob_v.at[p]
        r0p = r0_v.at[p]
        r1p = r1_v.at[p]

        def tok(j, carry):
            lane = jnp.full((16,), j, jnp.int32)
            s0 = _take16(wv0, lane)
            s1 = _take16(wv1, lane)
            for v in range(DM // 16):
                sl = pl.ds(v * 16, 16)
                obp[j, sl] = s0 * r0p[j, sl] + s1 * r1p[j, sl]
            return carry

        lax.fori_loop(0, ck, tok, jnp.int32(0))
        st[p] = pltpu.async_copy(obp, out_hbm.at[pl.ds(t0 + cj * ck, ck)],
                                 stsem[p])
    st[0].wait()
    st[1].wait()


def _combine(ys, pos, wts):
    mesh = plsc.VectorSubcoreMesh(core_axis_name="c", subcore_axis_name="s")
    fn = pl.kernel(
        _combine_body,
        out_type=jax.ShapeDtypeStruct((T, DM), jnp.float32),
        mesh=mesh,
        scratch_types=[
            pltpu.VMEM((TD,), jnp.int32),
            pltpu.VMEM((TD,), jnp.int32),
            pltpu.VMEM((TD,), jnp.float32),
            pltpu.VMEM((TD,), jnp.float32),
            pltpu.VMEM((2, 16, DM), jnp.float32),
            pltpu.VMEM((2, 16, DM), jnp.float32),
            pltpu.VMEM((2, 16, DM), jnp.float32),
            pltpu.SemaphoreType.DMA,
            pltpu.SemaphoreType.DMA,
            pltpu.SemaphoreType.DMA,
            pltpu.SemaphoreType.DMA,
            pltpu.SemaphoreType.DMA,
            pltpu.SemaphoreType.DMA,
        ],
    )
    return fn(ys, pos, wts)


def kernel(x, Wg, W1, b1, W2, b2, gamma, beta):
    Bsz, Slen, D = x.shape
    xf = x.reshape(T, DM)
    eids, wts = _gating(xf, Wg)
    xs, pos, bmap, bval = _dispatch(eids, xf)
    ys = _ffn(xs, bmap, bval, W1, b1, W2, b2, gamma, beta)
    out = _combine(ys, pos, wts)
    return out.reshape(Bsz, Slen, D)


# speedup vs baseline: 1.3222x; 1.0027x over previous
import jax
import jax.numpy as jnp
from jax import lax
from jax.experimental import pallas as pl
from jax.experimental.pallas import tpu as pltpu
from jax.experimental.pallas import tpu_sc as plsc

T = 2048
DM = 1024
HD = 2048
E = 8
BLK = 256
SP = 6144
NB = SP // BLK
NC = 2
NS = 16
NW = NC * NS
CH = 128
TD = T // NW


def _gating_body(x_ref, wg_ref, eids_ref, wts_ref):
    xb = x_ref[...]
    lt = jax.lax.dot_general(
        wg_ref[...], xb, (((1,), (1,)), ((), ())),
        preferred_element_type=jnp.float32)
    eidx = jax.lax.broadcasted_iota(jnp.int32, lt.shape, 0)
    m1 = jnp.max(lt, axis=0, keepdims=True)
    i1 = jnp.min(jnp.where(lt == m1, eidx, E), axis=0, keepdims=True)
    l2 = jnp.where(eidx == i1, -jnp.inf, lt)
    m2 = jnp.max(l2, axis=0, keepdims=True)
    i2 = jnp.min(jnp.where(l2 == m2, eidx, E), axis=0, keepdims=True)
    p = jnp.exp(m2 - m1)
    w1g = 1.0 / (1.0 + p)
    w2g = p / (1.0 + p)
    eids_ref[...] = jnp.concatenate([i1, i2], axis=0)
    wts_ref[...] = jnp.concatenate([w1g, w2g], axis=0)


def _gating(xf, Wg):
    blk = 256
    return pl.pallas_call(
        _gating_body,
        grid=(T // blk,),
        in_specs=[
            pl.BlockSpec((blk, DM), lambda tb: (tb, 0)),
            pl.BlockSpec((E, DM), lambda tb: (0, 0)),
        ],
        out_specs=(
            pl.BlockSpec((2, blk), lambda tb: (0, tb)),
            pl.BlockSpec((2, blk), lambda tb: (0, tb)),
        ),
        out_shape=(
            jax.ShapeDtypeStruct((2, T), jnp.int32),
            jax.ShapeDtypeStruct((2, T), jnp.float32),
        ),
    )(xf, Wg)


def _take16(vec, lane):
    dn = jax.lax.GatherDimensionNumbers(
        offset_dims=(), collapsed_slice_dims=(0,), start_index_map=(0,))
    return jax.lax.gather(
        vec, lane[:, None], dn, (1,),
        mode=jax.lax.GatherScatterMode.PROMISE_IN_BOUNDS)


def _allsum16(v):
    lane = lax.iota(jnp.int32, 16)
    for d in (8, 4, 2, 1):
        v = v + _take16(v, jnp.bitwise_xor(lane, d))
    return v


def _cumsum16(v):
    lane = lax.iota(jnp.int32, 16)
    for d in (1, 2, 4, 8):
        sh = _take16(v, jnp.maximum(lane - d, 0))
        v = v + jnp.where(lane >= d, sh, 0)
    return v


def _dispatch_body(eids_hbm, xf_hbm, xs_hbm, pos_hbm, bmap_hbm, bval_hbm,
                   eid_v, mych_v, posl_v, sidx_v, rows_v, bm_v, bv_v,
                   gsem0, gsem1, gsem2, ssem0, ssem1, ssem2):
    c = lax.axis_index("c")
    s = lax.axis_index("s")
    wid = s * NC + c
    k = wid // NS
    t0 = (wid % NS) * CH
    myvec0 = (wid % NS) * (CH // 16)
    nch = CH // 32
    gsem = (gsem0, gsem1, gsem2)
    ssem = (ssem0, ssem1, ssem2)
    g = [None] * nch
    sc = [None] * nch

    for cj in range(min(3, nch)):
        g[cj] = pltpu.async_copy(
            xf_hbm.at[pl.ds(t0 + cj * 32, 32)], rows_v.at[cj % 3],
            gsem[cj % 3])

    pltpu.sync_copy(eids_hbm, eid_v)
    pltpu.sync_copy(eids_hbm.at[k, pl.ds(t0, CH)], mych_v)

    zeros8 = tuple(jnp.zeros((16,), jnp.int32) for _ in range(E))

    def scan_row(row, lo, hi, acc):
        def body(i, a):
            v = eid_v[row, pl.ds(i * 16, 16)]
            return tuple(a[e] + jnp.where(v == e, 1, 0) for e in range(E))
        return lax.fori_loop(lo, hi, body, acc)

    nvr = T // 16
    b0 = jnp.where(k == 0, myvec0, nvr)
    b1 = jnp.where(k == 0, 0, myvec0)
    pre = scan_row(0, 0, b0, zeros8)
    pre = scan_row(1, 0, b1, pre)
    tot = scan_row(0, b0, nvr, pre)
    tot = scan_row(1, b1, nvr, tot)

    pre_c = [_allsum16(pre[e]) for e in range(E)]
    tot_c = [_allsum16(tot[e]) for e in range(E)]

    poff = []
    nblk = []
    run = jnp.zeros((16,), jnp.int32)
    for e in range(E):
        poff.append(run)
        nb_e = jnp.right_shift(tot_c[e] + (BLK - 1), 8)
        nblk.append(nb_e)
        run = run + jnp.left_shift(nb_e, 8)

    rcnt = [poff[e] + pre_c[e] for e in range(E)]
    for j in range(CH // 16):
        v = mych_v[pl.ds(j * 16, 16)]
        pos_vec = jnp.zeros((16,), jnp.int32)
        for e in range(E):
            m = v == e
            m32 = jnp.where(m, 1, 0)
            cs = _cumsum16(m32)
            pos_vec = jnp.where(m, rcnt[e] + cs - 1, pos_vec)
            rcnt[e] = rcnt[e] + _allsum16(m32)
        posl_v[pl.ds(j * 16, 16)] = pos_vec
        sidx_v[j // 2, pl.ds((j % 2) * 16, 16)] = pos_vec

    pltpu.sync_copy(posl_v, pos_hbm.at[k, pl.ds(t0, CH)])

    @pl.when(wid == 0)
    def _():
        for half in range(2):
            jv = lax.iota(jnp.int32, 16) + half * 16
            bstart = jv * BLK
            bmv = jnp.full((16,), E - 1, jnp.int32)
            bvv = jnp.zeros((16,), jnp.int32)
            for e in range(E):
                inb = (bstart >= poff[e]) & (bstart < poff[e] + nblk[e] * BLK)
                bmv = jnp.where(inb, e, bmv)
                bvv = jnp.where(inb, 1, bvv)
            bm_v[pl.ds(half * 16, 16)] = bmv
            bv_v[pl.ds(half * 16, 16)] = bvv
        pltpu.sync_copy(bm_v, bmap_hbm)
        pltpu.sync_copy(bv_v, bval_hbm)

    waited = [False] * nch
    for cj in range(nch):
        b = cj % 3
        g[cj].wait()
        sc[cj] = pltpu.async_copy(rows_v.at[b], xs_hbm.at[sidx_v.at[cj]],
                                  ssem[b])
        if cj + 3 < nch:
            sc[cj].wait()
            waited[cj] = True
            g[cj + 3] = pltpu.async_copy(
                xf_hbm.at[pl.ds(t0 + (cj + 3) * 32, 32)], rows_v.at[b],
                gsem[b])
    for cj in range(nch):
        if not waited[cj]:
            sc[cj].wait()


def _dispatch(eids, xf):
    mesh = plsc.VectorSubcoreMesh(core_axis_name="c", subcore_axis_name="s")
    fn = pl.kernel(
        _dispatch_body,
        out_type=(
            jax.ShapeDtypeStruct((SP, DM), jnp.float32),
            jax.ShapeDtypeStruct((2, T), jnp.int32),
            jax.ShapeDtypeStruct((32,), jnp.int32),
            jax.ShapeDtypeStruct((32,), jnp.int32),
        ),
        mesh=mesh,
        scratch_types=[
            pltpu.VMEM((2, T), jnp.int32),
            pltpu.VMEM((CH,), jnp.int32),
            pltpu.VMEM((CH,), jnp.int32),
            pltpu.VMEM((CH // 32, 32), jnp.int32),
            pltpu.VMEM((3, 32, DM), jnp.float32),
            pltpu.VMEM((32,), jnp.int32),
            pltpu.VMEM((32,), jnp.int32),
            pltpu.SemaphoreType.DMA,
            pltpu.SemaphoreType.DMA,
            pltpu.SemaphoreType.DMA,
            pltpu.SemaphoreType.DMA,
            pltpu.SemaphoreType.DMA,
            pltpu.SemaphoreType.DMA,
        ],
    )
    return fn(eids, xf)


def _ffn_body(bmap_ref, bval_ref, xs_ref, w1_ref, b1_ref, w2_ref, b2_ref,
              g_ref, be_ref, ys_ref):
    b = pl.program_id(0)

    @pl.when(bval_ref[b] != 0)
    def _():
        xb = xs_ref[...]
        h = jnp.maximum(
            jnp.dot(xb, w1_ref[0], preferred_element_type=jnp.float32)
            + b1_ref[0], 0.0)
        y = jnp.dot(h, w2_ref[0], preferred_element_type=jnp.float32) \
            + b2_ref[0]
        mu = jnp.mean(y, axis=1, keepdims=True)
        yc = y - mu
        var = jnp.mean(yc * yc, axis=1, keepdims=True)
        ys_ref[...] = yc * jax.lax.rsqrt(var + 1e-5) * g_ref[0] + be_ref[0]


def _ffn(xs, bmap, bval, W1, b1, W2, b2, gamma, beta):
    grid_spec = pltpu.PrefetchScalarGridSpec(
        num_scalar_prefetch=2,
        grid=(NB,),
        in_specs=[
            pl.BlockSpec((BLK, DM), lambda b, bm, bv: (b * bv[b], 0)),
            pl.BlockSpec((1, DM, HD), lambda b, bm, bv: (bm[b], 0, 0)),
            pl.BlockSpec((1, 1, HD), lambda b, bm, bv: (bm[b], 0, 0)),
            pl.BlockSpec((1, HD, DM), lambda b, bm, bv: (bm[b], 0, 0)),
            pl.BlockSpec((1, 1, DM), lambda b, bm, bv: (bm[b], 0, 0)),
            pl.BlockSpec((1, 1, DM), lambda b, bm, bv: (bm[b], 0, 0)),
            pl.BlockSpec((1, 1, DM), lambda b, bm, bv: (bm[b], 0, 0)),
        ],
        out_specs=pl.BlockSpec(
            (BLK, DM),
            lambda b, bm, bv: (jnp.where(bv[b] != 0, b, NB - 1), 0)),
    )
    return pl.pallas_call(
        _ffn_body,
        grid_spec=grid_spec,
        out_shape=jax.ShapeDtypeStruct((SP, DM), jnp.float32),
        compiler_params=pltpu.CompilerParams(
            vmem_limit_bytes=100 * 1024 * 1024),
    )(bmap, bval, xs, W1, b1.reshape(E, 1, HD), W2, b2.reshape(E, 1, DM),
      gamma.reshape(E, 1, DM), beta.reshape(E, 1, DM))


def _take16(vec, lane):
    dn = jax.lax.GatherDimensionNumbers(
        offset_dims=(), collapsed_slice_dims=(0,), start_index_map=(0,))
    return jax.lax.gather(
        vec, lane[:, None], dn, (1,),
        mode=jax.lax.GatherScatterMode.PROMISE_IN_BOUNDS)

def _combine_body(ys_hbm, pos_hbm, wts_hbm, out_hbm,
                  p0_v, p1_v, w0_v, w1_v, r0_v, r1_v, ob_v,
                  g0s0, g0s1, g1s0, g1s1, sts0, sts1):
    c = lax.axis_index("c")
    s = lax.axis_index("s")
    wid = s * NC + c
    t0 = wid * TD
    ck = 16
    nch = TD // ck

    pltpu.sync_copy(pos_hbm.at[0, pl.ds(t0, TD)], p0_v)
    pltpu.sync_copy(pos_hbm.at[1, pl.ds(t0, TD)], p1_v)

    g0sem = (g0s0, g0s1)
    g1sem = (g1s0, g1s1)
    stsem = (sts0, sts1)
    g0 = [None, None]
    g1 = [None, None]
    st = [None, None]

    def gstart(cj, p):
        g0[p] = pltpu.async_copy(
            ys_hbm.at[p0_v.at[pl.ds(cj * ck, ck)]], r0_v.at[p], g0sem[p])
        g1[p] = pltpu.async_copy(
            ys_hbm.at[p1_v.at[pl.ds(cj * ck, ck)]], r1_v.at[p], g1sem[p])

    gstart(0, 0)
    pltpu.sync_copy(wts_hbm.at[0, pl.ds(t0, TD)], w0_v)
    pltpu.sync_copy(wts_hbm.at[1, pl.ds(t0, TD)], w1_v)
    for cj in range(nch):
        p = cj % 2
        g0[p].wait()
        g1[p].wait()
        if cj + 1 < nch:
            gstart(cj + 1, (cj + 1) % 2)
        if st[p] is not None:
            st[p].wait()

        wv0 = w0_v[pl.ds(cj * ck, 16)]
        wv1 = w1_v[pl.ds(cj * ck, 16)]
        obp = ob_v.at[p]
        r0p = r0_v.at[p]
        r1p = r1_v.at[p]

        def tok(j, carry):
            lane = jnp.full((16,), j, jnp.int32)
            s0 = _take16(wv0, lane)
            s1 = _take16(wv1, lane)
            for v in range(DM // 16):
                sl = pl.ds(v * 16, 16)
                obp[j, sl] = s0 * r0p[j, sl] + s1 * r1p[j, sl]
            return carry

        lax.fori_loop(0, ck, tok, jnp.int32(0))
        st[p] = pltpu.async_copy(obp, out_hbm.at[pl.ds(t0 + cj * ck, ck)],
                                 stsem[p])
    st[0].wait()
    st[1].wait()


def _combine(ys, pos, wts):
    mesh = plsc.VectorSubcoreMesh(core_axis_name="c", subcore_axis_name="s")
    fn = pl.kernel(
        _combine_body,
        out_type=jax.ShapeDtypeStruct((T, DM), jnp.float32),
        mesh=mesh,
        scratch_types=[
            pltpu.VMEM((TD,), jnp.int32),
            pltpu.VMEM((TD,), jnp.int32),
            pltpu.VMEM((TD,), jnp.float32),
            pltpu.VMEM((TD,), jnp.float32),
            pltpu.VMEM((2, 16, DM), jnp.float32),
            pltpu.VMEM((2, 16, DM), jnp.float32),
            pltpu.VMEM((2, 16, DM), jnp.float32),
            pltpu.SemaphoreType.DMA,
            pltpu.SemaphoreType.DMA,
            pltpu.SemaphoreType.DMA,
            pltpu.SemaphoreType.DMA,
            pltpu.SemaphoreType.DMA,
            pltpu.SemaphoreType.DMA,
        ],
    )
    return fn(ys, pos, wts)


def kernel(x, Wg, W1, b1, W2, b2, gamma, beta):
    Bsz, Slen, D = x.shape
    xf = x.reshape(T, DM)
    eids, wts = _gating(xf, Wg)
    xs, pos, bmap, bval = _dispatch(eids, xf)
    ys = _ffn(xs, bmap, bval, W1, b1, W2, b2, gamma, beta)
    out = _combine(ys, pos, wts)
    return out.reshape(Bsz, Slen, D)
